# baseline (speedup 1.0000x reference)
import jax
import jax.numpy as jnp
from jax.experimental import pallas as pl

_B, _P, _Q, _A, _KP, _KQ = 8, 30, 9000, 4, 3, 3
_QT = 3072
_NV = _P + 1
_QK = _Q // _KQ
_PK = _P // _KP


def _row(x_col):
    i = jax.lax.broadcasted_iota(jnp.int32, (_P, _P), 0)
    j = jax.lax.broadcasted_iota(jnp.int32, (_P, _P), 1)
    eye = (i == j).astype(jnp.float32)
    return jnp.sum(eye * x_col, axis=0, keepdims=True)


def _fused_kernel(
    m_ref, gt_ref, w0a_ref, w0b_ref, b0_ref,
    w1_ref, b1_ref, w2_ref, b2_ref, w3_ref, b3_ref,
    rw0_ref, rb0_ref, rw1_ref, rb1_ref, rw2_ref, rb2_ref, rw3_ref, rb3_ref,
    cor_ref, ref_ref,
):
    b = pl.program_id(0)

    @pl.when(b == 0)
    def _init():
        cor_ref[...] = jnp.zeros((1, 1), jnp.float32)
        ref_ref[...] = jnp.zeros((1, 1), jnp.float32)

    m = m_ref[0]
    gt = gt_ref[0]

    ones_row = jnp.ones((1, _P), jnp.float32)
    cnts = [
        jnp.dot(ones_row, (m == a).astype(jnp.float32),
                preferred_element_type=jnp.float32)
        for a in range(_A - 1)
    ]
    cnts.append(float(_P) - cnts[0] - cnts[1] - cnts[2])
    key = cnts[0] * float(_A) + float(_A - 1)
    for a in range(1, _A):
        key = jnp.maximum(key, cnts[a] * float(_A) + float(_A - 1 - a))
    g = (_A - 1) - jnp.mod(key.astype(jnp.int32), _A)
    mc = (m == g).astype(jnp.float32)
    a_cnt = jnp.sum(mc, axis=1, keepdims=True)
    d_cnt = jnp.dot(ones_row, mc, preferred_element_type=jnp.float32)

    vi = jax.lax.broadcasted_iota(jnp.int32, (_NV, 1), 0).astype(jnp.float32)
    cmp = (vi >= jnp.broadcast_to(d_cnt, (_NV, _Q))).astype(jnp.float32)
    ih = jnp.sum(cmp, axis=1, keepdims=True)
    b1, b2 = float(_QK), float(2 * _QK)
    v1 = jnp.sum((ih <= b1).astype(jnp.float32))
    v2 = jnp.sum((ih <= b2).astype(jnp.float32))
    less1 = jnp.sum((d_cnt < v1).astype(jnp.float32))
    less2 = jnp.sum((d_cnt < v2).astype(jnp.float32))
    ind1 = (d_cnt == v1).astype(jnp.float32)
    ind2 = (d_cnt == v2).astype(jnp.float32)
    incl = jnp.concatenate([ind1, ind2], axis=0)
    s = 1
    while s < _Q:
        z = jnp.zeros((2, s), jnp.float32)
        incl = incl + jnp.concatenate([z, incl[:, : _Q - s]], axis=1)
        s *= 2
    pos1 = incl[0:1, :] - ind1
    pos2 = incl[1:2, :] - ind2
    ge1 = (d_cnt > v1) | ((d_cnt == v1) & (less1 + pos1 >= b1))
    ge2 = (d_cnt > v2) | ((d_cnt == v2) & (less2 + pos2 >= b2))
    tm = [~ge1, ge1 & ~ge2, ge2]
    a_k_rows = []
    for k in range(_KQ):
        ak_col = jnp.sum(mc * tm[k].astype(jnp.float32), axis=1, keepdims=True)
        a_k_rows.append(_row(ak_col / float(_QK)))

    acf = a_cnt
    a_row = _row(acf)
    ii = jax.lax.broadcasted_iota(jnp.int32, (_P, _P), 0)
    jj = jax.lax.broadcasted_iota(jnp.int32, (_P, _P), 1)
    less_m = (a_row < acf).astype(jnp.float32)
    eq_m = ((a_row == acf) & (jj < ii)).astype(jnp.float32)
    rank_p = jnp.sum(less_m + eq_m, axis=1, keepdims=True)
    srows = []
    for k in range(_KP):
        gk = (rank_p >= float(k * _PK)) & (rank_p < float((k + 1) * _PK))
        srows.append(_row(gk.astype(jnp.float32)))
    sel = jnp.concatenate(srows, axis=0)
    d_k = jnp.dot(sel, mc, preferred_element_type=jnp.float32) / float(_PK)

    f_d = jnp.concatenate([d_cnt / float(_P), d_k], axis=0)
    f_at = jnp.concatenate([a_row / float(_Q)] + a_k_rows, axis=0)
    apt = jnp.dot(w0a_ref[...], f_at, preferred_element_type=jnp.float32)
    dq = (
        jnp.dot(w0b_ref[...], f_d, preferred_element_type=jnp.float32)
        + b0_ref[...]
    )

    cor_acc = jnp.zeros((1, 1), jnp.float32)
    ref_acc = jnp.zeros((1, 1), jnp.float32)
    for k0 in range(0, _Q, _QT):
        w = min(_QT, _Q - k0)
        mk = m[:, k0 : k0 + w]
        gtk = gt[:, k0 : k0 + w]
        dqk = dq[:, k0 : k0 + w]

        h = jax.nn.relu(
            jnp.concatenate([apt[:, p : p + 1] + dqk for p in range(_P)], axis=1)
        )
        h = jax.nn.relu(
            jnp.dot(w1_ref[...], h, preferred_element_type=jnp.float32)
            + b1_ref[...]
        )
        h = jax.nn.relu(
            jnp.dot(w2_ref[...], h, preferred_element_type=jnp.float32)
            + b2_ref[...]
        )
        y = jax.nn.sigmoid(
            jnp.dot(w3_ref[...], h, preferred_element_type=jnp.float32)
            + b3_ref[...]
        )
        yp = jnp.concatenate(
            [y[:, p * w : (p + 1) * w] for p in range(_P)], axis=0
        )

        log_y = jnp.maximum(jnp.log(yp), -100.0)
        log_1my = jnp.maximum(jnp.log(1.0 - yp), -100.0)
        bce = jnp.where(mk == gtk, log_y, log_1my)
        cor_acc = cor_acc + jnp.sum(bce, keepdims=True)

        other = (1.0 - yp) / float(_A - 1)
        u4 = jnp.concatenate(
            [jnp.where(mk == a, yp, other) for a in range(_A)], axis=1
        )
        z = jnp.tanh(
            jnp.dot(rw0_ref[...], u4, preferred_element_type=jnp.float32)
            + rb0_ref[...]
        )
        z = jnp.tanh(
            jnp.dot(rw1_ref[...], z, preferred_element_type=jnp.float32)
            + rb1_ref[...]
        )
        z = jnp.tanh(
            jnp.dot(rw2_ref[...], z, preferred_element_type=jnp.float32)
            + rb2_ref[...]
        )
        s4 = (
            jnp.dot(rw3_ref[...], z, preferred_element_type=jnp.float32)
            + rb3_ref[...]
        )
        scores = jnp.concatenate(
            [s4[:, a * w : (a + 1) * w] for a in range(_A)], axis=0
        )
        mx = jnp.max(scores, axis=0, keepdims=True)
        e = jnp.exp(scores - mx)
        probs = e / jnp.sum(e, axis=0, keepdims=True)
        mx2 = jnp.max(probs, axis=0, keepdims=True)
        lse2 = jnp.log(jnp.sum(jnp.exp(probs - mx2), axis=0, keepdims=True))
        logp = probs - mx2 - lse2
        toh = jax.lax.broadcasted_iota(jnp.int32, (_A, w), 0) == gtk
        ref_acc = ref_acc + jnp.sum(jnp.where(toh, logp, 0.0), keepdims=True)

    cor_ref[...] += cor_acc
    ref_ref[...] += ref_acc

    @pl.when(b == _B - 1)
    def _fin():
        cor_ref[...] = -cor_ref[...] / float(_B * _P * _Q)
        ref_ref[...] = -ref_ref[...] / float(_B * _Q)


def kernel(M, G_true, cW0, cb0, cW1, cb1, cW2, cb2, cW3, cb3,
           rW0, rb0, rW1, rb1, rW2, rb2, rW3, rb3):
    mi = M.astype(jnp.int32)
    gt3 = G_true.astype(jnp.int32).reshape(_B, 1, _Q)
    full = lambda shape: pl.BlockSpec(shape, lambda b: (0, 0))
    cor, refl = pl.pallas_call(
        _fused_kernel,
        grid=(_B,),
        in_specs=[
            pl.BlockSpec((1, _P, _Q), lambda b: (b, 0, 0)),
            pl.BlockSpec((1, 1, _Q), lambda b: (b, 0, 0)),
            full((10, 1 + _KQ)), full((10, 1 + _KP)), full((10, 1)),
            full((10, 10)), full((10, 1)),
            full((10, 10)), full((10, 1)),
            full((1, 10)), full((1, 1)),
            full((15, _P)), full((15, 1)),
            full((15, 15)), full((15, 1)),
            full((15, 15)), full((15, 1)),
            full((1, 15)), full((1, 1)),
        ],
        out_specs=[
            pl.BlockSpec((1, 1), lambda b: (0, 0)),
            pl.BlockSpec((1, 1), lambda b: (0, 0)),
        ],
        out_shape=[
            jax.ShapeDtypeStruct((1, 1), jnp.float32),
            jax.ShapeDtypeStruct((1, 1), jnp.float32),
        ],
    )(
        mi, gt3,
        cW0[:, : 1 + _KQ], cW0[:, 1 + _KQ :], cb0.reshape(10, 1),
        cW1, cb1.reshape(10, 1), cW2, cb2.reshape(10, 1),
        cW3.reshape(1, 10), cb3.reshape(1, 1),
        rW0, rb0.reshape(15, 1), rW1, rb1.reshape(15, 1),
        rW2, rb2.reshape(15, 1), rW3.reshape(1, 15), rb3.reshape(1, 1),
    )
    return cor[0, 0], refl[0, 0]


# speedup vs baseline: 1.0067x; 1.0067x over previous
import jax
import jax.numpy as jnp
from jax.experimental import pallas as pl

_B, _P, _Q, _A, _KP, _KQ = 8, 30, 9000, 4, 3, 3
_QT = 2304
_NV = _P + 1
_QK = _Q // _KQ
_PK = _P // _KP


def _row(x_col):
    i = jax.lax.broadcasted_iota(jnp.int32, (_P, _P), 0)
    j = jax.lax.broadcasted_iota(jnp.int32, (_P, _P), 1)
    eye = (i == j).astype(jnp.float32)
    return jnp.sum(eye * x_col, axis=0, keepdims=True)


def _fused_kernel(
    m_ref, gt_ref, w0a_ref, w0b_ref, b0_ref,
    w1_ref, b1_ref, w2_ref, b2_ref, w3_ref, b3_ref,
    rw0_ref, rb0_ref, rw1_ref, rb1_ref, rw2_ref, rb2_ref, rw3_ref, rb3_ref,
    cor_ref, ref_ref,
):
    b = pl.program_id(0)

    @pl.when(b == 0)
    def _init():
        cor_ref[...] = jnp.zeros((1, 1), jnp.float32)
        ref_ref[...] = jnp.zeros((1, 1), jnp.float32)

    m = m_ref[0]
    gt = gt_ref[0]

    ones_row = jnp.ones((1, _P), jnp.float32)
    cnts = [
        jnp.dot(ones_row, (m == a).astype(jnp.float32),
                preferred_element_type=jnp.float32)
        for a in range(_A - 1)
    ]
    cnts.append(float(_P) - cnts[0] - cnts[1] - cnts[2])
    key = cnts[0] * float(_A) + float(_A - 1)
    for a in range(1, _A):
        key = jnp.maximum(key, cnts[a] * float(_A) + float(_A - 1 - a))
    g = (_A - 1) - jnp.mod(key.astype(jnp.int32), _A)
    mc = (m == g).astype(jnp.float32)
    a_cnt = jnp.sum(mc, axis=1, keepdims=True)
    d_cnt = jnp.dot(ones_row, mc, preferred_element_type=jnp.float32)

    vi = jax.lax.broadcasted_iota(jnp.int32, (_NV, 1), 0).astype(jnp.float32)
    cmp = (vi >= jnp.broadcast_to(d_cnt, (_NV, _Q))).astype(jnp.float32)
    ih = jnp.sum(cmp, axis=1, keepdims=True)
    b1, b2 = float(_QK), float(2 * _QK)
    v1 = jnp.sum((ih <= b1).astype(jnp.float32))
    v2 = jnp.sum((ih <= b2).astype(jnp.float32))
    less1 = jnp.sum((d_cnt < v1).astype(jnp.float32))
    less2 = jnp.sum((d_cnt < v2).astype(jnp.float32))
    ind1 = (d_cnt == v1).astype(jnp.float32)
    ind2 = (d_cnt == v2).astype(jnp.float32)
    incl = jnp.concatenate([ind1, ind2], axis=0)
    s = 1
    while s < _Q:
        z = jnp.zeros((2, s), jnp.float32)
        incl = incl + jnp.concatenate([z, incl[:, : _Q - s]], axis=1)
        s *= 2
    pos1 = incl[0:1, :] - ind1
    pos2 = incl[1:2, :] - ind2
    ge1 = (d_cnt > v1) | ((d_cnt == v1) & (less1 + pos1 >= b1))
    ge2 = (d_cnt > v2) | ((d_cnt == v2) & (less2 + pos2 >= b2))
    tm = [~ge1, ge1 & ~ge2, ge2]
    a_k_rows = []
    for k in range(_KQ):
        ak_col = jnp.sum(mc * tm[k].astype(jnp.float32), axis=1, keepdims=True)
        a_k_rows.append(_row(ak_col / float(_QK)))

    acf = a_cnt
    a_row = _row(acf)
    ii = jax.lax.broadcasted_iota(jnp.int32, (_P, _P), 0)
    jj = jax.lax.broadcasted_iota(jnp.int32, (_P, _P), 1)
    less_m = (a_row < acf).astype(jnp.float32)
    eq_m = ((a_row == acf) & (jj < ii)).astype(jnp.float32)
    rank_p = jnp.sum(less_m + eq_m, axis=1, keepdims=True)
    srows = []
    for k in range(_KP):
        gk = (rank_p >= float(k * _PK)) & (rank_p < float((k + 1) * _PK))
        srows.append(_row(gk.astype(jnp.float32)))
    sel = jnp.concatenate(srows, axis=0)
    d_k = jnp.dot(sel, mc, preferred_element_type=jnp.float32) / float(_PK)

    f_d = jnp.concatenate([d_cnt / float(_P), d_k], axis=0)
    f_at = jnp.concatenate([a_row / float(_Q)] + a_k_rows, axis=0)
    apt = jnp.dot(w0a_ref[...], f_at, preferred_element_type=jnp.float32)
    dq = (
        jnp.dot(w0b_ref[...], f_d, preferred_element_type=jnp.float32)
        + b0_ref[...]
    )

    cor_acc = jnp.zeros((1, 1), jnp.float32)
    ref_acc = jnp.zeros((1, 1), jnp.float32)
    for k0 in range(0, _Q, _QT):
        w = min(_QT, _Q - k0)
        mk = m[:, k0 : k0 + w]
        gtk = gt[:, k0 : k0 + w]
        dqk = dq[:, k0 : k0 + w]

        ap_rep = jnp.concatenate(
            [jnp.broadcast_to(apt[:, p : p + 1], (10, w)) for p in range(_P)],
            axis=1,
        )
        dq_tiled = jnp.concatenate([dqk] * _P, axis=1)
        h = jax.nn.relu(ap_rep + dq_tiled)
        h = jax.nn.relu(
            jnp.dot(w1_ref[...], h, preferred_element_type=jnp.float32)
            + b1_ref[...]
        )
        h = jax.nn.relu(
            jnp.dot(w2_ref[...], h, preferred_element_type=jnp.float32)
            + b2_ref[...]
        )
        y = jax.nn.sigmoid(
            jnp.dot(w3_ref[...], h, preferred_element_type=jnp.float32)
            + b3_ref[...]
        )
        yp = jnp.concatenate(
            [y[:, p * w : (p + 1) * w] for p in range(_P)], axis=0
        )

        log_y = jnp.maximum(jnp.log(yp), -100.0)
        log_1my = jnp.maximum(jnp.log(1.0 - yp), -100.0)
        bce = jnp.where(mk == gtk, log_y, log_1my)
        cor_acc = cor_acc + jnp.sum(bce, keepdims=True)

        other = (1.0 - yp) / float(_A - 1)
        u4 = jnp.concatenate(
            [jnp.where(mk == a, yp, other) for a in range(_A)], axis=1
        )
        z = jnp.tanh(
            jnp.dot(rw0_ref[...], u4, preferred_element_type=jnp.float32)
            + rb0_ref[...]
        )
        z = jnp.tanh(
            jnp.dot(rw1_ref[...], z, preferred_element_type=jnp.float32)
            + rb1_ref[...]
        )
        z = jnp.tanh(
            jnp.dot(rw2_ref[...], z, preferred_element_type=jnp.float32)
            + rb2_ref[...]
        )
        s4 = (
            jnp.dot(rw3_ref[...], z, preferred_element_type=jnp.float32)
            + rb3_ref[...]
        )
        scores = jnp.concatenate(
            [s4[:, a * w : (a + 1) * w] for a in range(_A)], axis=0
        )
        mx = jnp.max(scores, axis=0, keepdims=True)
        e = jnp.exp(scores - mx)
        probs = e / jnp.sum(e, axis=0, keepdims=True)
        mx2 = jnp.max(probs, axis=0, keepdims=True)
        lse2 = jnp.log(jnp.sum(jnp.exp(probs - mx2), axis=0, keepdims=True))
        logp = probs - mx2 - lse2
        toh = jax.lax.broadcasted_iota(jnp.int32, (_A, w), 0) == gtk
        ref_acc = ref_acc + jnp.sum(jnp.where(toh, logp, 0.0), keepdims=True)

    cor_ref[...] += cor_acc
    ref_ref[...] += ref_acc

    @pl.when(b == _B - 1)
    def _fin():
        cor_ref[...] = -cor_ref[...] / float(_B * _P * _Q)
        ref_ref[...] = -ref_ref[...] / float(_B * _Q)


def kernel(M, G_true, cW0, cb0, cW1, cb1, cW2, cb2, cW3, cb3,
           rW0, rb0, rW1, rb1, rW2, rb2, rW3, rb3):
    mi = M.astype(jnp.int32)
    gt3 = G_true.astype(jnp.int32).reshape(_B, 1, _Q)
    full = lambda shape: pl.BlockSpec(shape, lambda b: (0, 0))
    cor, refl = pl.pallas_call(
        _fused_kernel,
        grid=(_B,),
        in_specs=[
            pl.BlockSpec((1, _P, _Q), lambda b: (b, 0, 0)),
            pl.BlockSpec((1, 1, _Q), lambda b: (b, 0, 0)),
            full((10, 1 + _KQ)), full((10, 1 + _KP)), full((10, 1)),
            full((10, 10)), full((10, 1)),
            full((10, 10)), full((10, 1)),
            full((1, 10)), full((1, 1)),
            full((15, _P)), full((15, 1)),
            full((15, 15)), full((15, 1)),
            full((15, 15)), full((15, 1)),
            full((1, 15)), full((1, 1)),
        ],
        out_specs=[
            pl.BlockSpec((1, 1), lambda b: (0, 0)),
            pl.BlockSpec((1, 1), lambda b: (0, 0)),
        ],
        out_shape=[
            jax.ShapeDtypeStruct((1, 1), jnp.float32),
            jax.ShapeDtypeStruct((1, 1), jnp.float32),
        ],
    )(
        mi, gt3,
        cW0[:, : 1 + _KQ], cW0[:, 1 + _KQ :], cb0.reshape(10, 1),
        cW1, cb1.reshape(10, 1), cW2, cb2.reshape(10, 1),
        cW3.reshape(1, 10), cb3.reshape(1, 1),
        rW0, rb0.reshape(15, 1), rW1, rb1.reshape(15, 1),
        rW2, rb2.reshape(15, 1), rW3.reshape(1, 15), rb3.reshape(1, 1),
    )
    return cor[0, 0], refl[0, 0]
